# baseline (speedup 1.0000x reference)
import functools

import jax
import jax.numpy as jnp
from jax import lax
from jax.experimental import pallas as pl
from jax.experimental.pallas import tpu as pltpu
from jax.experimental.pallas import tpu_sc as plsc

VOCAB = 100000
EMBED_DIM = 32
BATCH = 1024
CTX_LEN = 20

NUM_WORKERS = 32
B_PER_W = BATCH // NUM_WORKERS
IDX_PER_W = B_PER_W * CTX_LEN
GATHER_CHUNK = 128
N_CHUNKS = IDX_PER_W // GATHER_CHUNK


def _sc_gather_mean_t(context_t, emb_t):
    mesh = plsc.VectorSubcoreMesh(core_axis_name="c", subcore_axis_name="s")

    @functools.partial(
        pl.kernel,
        mesh=mesh,
        out_type=jax.ShapeDtypeStruct((EMBED_DIM, BATCH), jnp.float32),
        scratch_types=[
            pltpu.VMEM((CTX_LEN, BATCH), jnp.int32),
            pltpu.VMEM((VOCAB,), jnp.float32),
            pltpu.VMEM((BATCH,), jnp.float32),
            pltpu.SemaphoreType.DMA,
            pltpu.SemaphoreType.DMA,
        ],
        compiler_params=pltpu.CompilerParams(
            use_tc_tiling_on_sc=True, needs_layout_passes=False
        ),
    )
    def gather_mean(ctx_hbm, embt_hbm, out_hbm, idx_v, row_v, acc_v, sem1, sem2):
        c = lax.axis_index("c")
        s = lax.axis_index("s")
        d = c * 16 + s
        c1 = pltpu.async_copy(ctx_hbm, idx_v, sem1)
        c2 = pltpu.async_copy(embt_hbm.at[d], row_v, sem2)
        c1.wait()
        c2.wait()

        @plsc.parallel_loop(0, BATCH, step=16, unroll=4)
        def body(b0):
            acc = jnp.zeros((16,), jnp.float32)
            for j in range(CTX_LEN):
                cidx = idx_v[j, pl.ds(b0, 16)]
                acc = acc + plsc.load_gather(row_v, [cidx])
            acc_v[pl.ds(b0, 16)] = acc * jnp.float32(1.0 / CTX_LEN)

        pltpu.sync_copy(acc_v, out_hbm.at[d])

    return gather_mean(context_t, emb_t)


V_BLOCK = 2048
N_VBLOCKS = (VOCAB + V_BLOCK - 1) // V_BLOCK


def _tc_matmul_kernel(wt_ref, avgt_ref, out_ref):
    out_ref[...] = lax.dot_general(
        wt_ref[...],
        avgt_ref[...],
        (((0,), (0,)), ((), ())),
        preferred_element_type=jnp.float32,
    )


def _tc_logits_t(WT, avgt):
    return pl.pallas_call(
        _tc_matmul_kernel,
        grid=(N_VBLOCKS,),
        in_specs=[
            pl.BlockSpec((EMBED_DIM, V_BLOCK), lambda i: (0, i)),
            pl.BlockSpec((EMBED_DIM, BATCH), lambda i: (0, 0)),
        ],
        out_specs=pl.BlockSpec((V_BLOCK, BATCH), lambda i: (i, 0)),
        out_shape=jax.ShapeDtypeStruct((VOCAB, BATCH), jnp.float32),
        compiler_params=pltpu.CompilerParams(
            dimension_semantics=("arbitrary",),
        ),
    )(WT, avgt)


def kernel(context, emb_table, W):
    context_t = context.astype(jnp.int32).T
    avgt = _sc_gather_mean_t(context_t, emb_table.T)
    return _tc_logits_t(W.T, avgt).T


# speedup vs baseline: 1.0064x; 1.0064x over previous
import functools

import jax
import jax.numpy as jnp
from jax import lax
from jax.experimental import pallas as pl
from jax.experimental.pallas import tpu as pltpu
from jax.experimental.pallas import tpu_sc as plsc

VOCAB = 100000
EMBED_DIM = 32
BATCH = 1024
CTX_LEN = 20

NUM_WORKERS = 32
B_PER_W = BATCH // NUM_WORKERS
IDX_PER_W = B_PER_W * CTX_LEN
GATHER_CHUNK = 128
N_CHUNKS = IDX_PER_W // GATHER_CHUNK


def _sc_gather_mean_t(context_t, emb_t):
    mesh = plsc.VectorSubcoreMesh(core_axis_name="c", subcore_axis_name="s")

    @functools.partial(
        pl.kernel,
        mesh=mesh,
        out_type=jax.ShapeDtypeStruct((EMBED_DIM, BATCH), jnp.float32),
        scratch_types=[
            pltpu.VMEM((CTX_LEN * BATCH,), jnp.int32),
            pltpu.VMEM((VOCAB,), jnp.float32),
            pltpu.VMEM((BATCH,), jnp.float32),
            pltpu.SemaphoreType.DMA,
            pltpu.SemaphoreType.DMA,
        ],
        compiler_params=pltpu.CompilerParams(
            use_tc_tiling_on_sc=True, needs_layout_passes=False
        ),
    )
    def gather_mean(ctx_hbm, embt_hbm, out_hbm, idx_v, row_v, acc_v, sem1, sem2):
        c = lax.axis_index("c")
        s = lax.axis_index("s")
        d = c * 16 + s
        c1 = pltpu.async_copy(ctx_hbm, idx_v, sem1)
        c2 = pltpu.async_copy(embt_hbm.at[d], row_v, sem2)
        c1.wait()
        c2.wait()

        @plsc.parallel_loop(0, BATCH, step=16, unroll=4)
        def body(b0):
            acc = jnp.zeros((16,), jnp.float32)
            for j in range(CTX_LEN):
                cidx = idx_v[pl.ds(j * BATCH + b0, 16)]
                acc = acc + plsc.load_gather(row_v, [cidx])
            acc_v[pl.ds(b0, 16)] = acc * jnp.float32(1.0 / CTX_LEN)

        pltpu.sync_copy(acc_v, out_hbm.at[d])

    return gather_mean(context_t, emb_t)


V_BLOCK = 2048
N_VBLOCKS = (VOCAB + V_BLOCK - 1) // V_BLOCK


def _tc_matmul_kernel(wt_ref, avgt_ref, out_ref):
    out_ref[...] = lax.dot_general(
        wt_ref[...],
        avgt_ref[...],
        (((0,), (0,)), ((), ())),
        preferred_element_type=jnp.float32,
    )


def _tc_logits_t(WT, avgt):
    return pl.pallas_call(
        _tc_matmul_kernel,
        grid=(N_VBLOCKS,),
        in_specs=[
            pl.BlockSpec((EMBED_DIM, V_BLOCK), lambda i: (0, i)),
            pl.BlockSpec((EMBED_DIM, BATCH), lambda i: (0, 0)),
        ],
        out_specs=pl.BlockSpec((V_BLOCK, BATCH), lambda i: (i, 0)),
        out_shape=jax.ShapeDtypeStruct((VOCAB, BATCH), jnp.float32),
        compiler_params=pltpu.CompilerParams(
            dimension_semantics=("arbitrary",),
        ),
    )(WT, avgt)


def kernel(context, emb_table, W):
    context_t = context.astype(jnp.int32).T.reshape(-1)
    avgt = _sc_gather_mean_t(context_t, emb_table.T)
    return _tc_logits_t(W.T, avgt).T
